# Initial kernel scaffold; baseline (speedup 1.0000x reference)
import jax
import jax.numpy as jnp
from jax.experimental import pallas as pl


def kernel(cls_logits, reg_deltas, fmap_h, fmap_w, img_h, img_w):
    raise NotImplementedError("write your pallas kernel here")


# speedup vs baseline: 66.5065x; 66.5065x over previous
import math

import jax
import jax.numpy as jnp
from jax import lax
from jax.experimental import pallas as pl
from jax.experimental.pallas import tpu as pltpu

_SCALES = [64.0, 128.0, 256.0, 512.0]
_RATIOS = [0.5, 1.0]
_NMS_THR = 0.7
_K_PRE = 1000
_K_POST = 300
_MIN_SIZE = 0.001
_BBOX_CLAMP = math.log(1000.0 / 16.0)

_N = 20000
_NPAD = 20480
_NROW = 160
_KP = 1024
_OUTP = 304


def _f32(x):
    return x.astype(jnp.float32)


def _iota2(shape, dim):
    return lax.broadcasted_iota(jnp.int32, shape, dim)


def _tr(x):
    n = x.shape[0]
    eye = _f32(_iota2((n, n), 0) == _iota2((n, n), 1))
    return lax.dot_general(x, eye, (((0,), (0,)), ((), ())),
                           preferred_element_type=jnp.float32, precision=lax.Precision.HIGHEST)


def _cumsum2d(x):
    u = _f32(_iota2((128, 128), 0) <= _iota2((128, 128), 1))
    rowc = lax.dot_general(x, u, (((1,), (0,)), ((), ())),
                           preferred_element_type=jnp.float32, precision=lax.Precision.HIGHEST)
    rt = rowc[:, 127:128]
    ls = _f32(_iota2((_NROW, _NROW), 1) < _iota2((_NROW, _NROW), 0))
    rpre = lax.dot_general(ls, rt, (((1,), (0,)), ((), ())),
                           preferred_element_type=jnp.float32, precision=lax.Precision.HIGHEST)
    return rowc + rpre


def _row2col(x_row):
    n = x_row.shape[1]
    diag = _f32(_iota2((n, n), 0) == _iota2((n, n), 1))
    return jnp.sum(x_row * diag, axis=1, keepdims=True)


def _body(score_ref, dx_ref, dy_ref, dw_ref, dh_ref,
          aw_ref, ah_ref, ax_ref, ay_ref, bounds_ref,
          out_ref, val_ref):
    f32 = jnp.float32
    s = score_ref[...]
    sbits = lax.bitcast_convert_type(s, jnp.int32)

    def bs_body(_, lohi):
        lo, hi = lohi
        mid = (lo + hi) // 2
        cnt = jnp.sum(jnp.where(sbits >= mid, 1.0, 0.0).astype(f32))
        feas = cnt >= float(_K_PRE)
        return (jnp.where(feas, mid, lo), jnp.where(feas, hi, mid))

    lo0 = jnp.int32(0)
    hi0 = jnp.int32(0x3F800001)
    vbits, _ = lax.fori_loop(0, 31, bs_body, (lo0, hi0))

    gt = sbits > vbits
    eq = sbits == vbits
    g = jnp.sum(jnp.where(gt, 1.0, 0.0).astype(f32))
    need = float(_K_PRE) - g
    eqc = _cumsum2d(jnp.where(eq, 1.0, 0.0).astype(f32))
    sel = gt | (eq & (eqc <= need))
    self_f = jnp.where(sel, 1.0, 0.0).astype(f32)
    pos0 = _cumsum2d(self_f) - 1.0
    posm = jnp.where(sel, pos0, 2.0e6)

    aw = aw_ref[...]
    ah = ah_ref[...]
    dwc = jnp.minimum(dw_ref[...], _BBOX_CLAMP)
    dhc = jnp.minimum(dh_ref[...], _BBOX_CLAMP)
    pcx = dx_ref[...] * aw + ax_ref[...]
    pcy = dy_ref[...] * ah + ay_ref[...]
    pw = jnp.exp(dwc) * aw
    ph = jnp.exp(dhc) * ah
    x1 = pcx - 0.5 * pw
    y1 = pcy - 0.5 * ph
    x2 = pcx + 0.5 * pw
    y2 = pcy + 0.5 * ph
    zrow = jnp.zeros((_NROW, 1, 128), f32)
    val_ref[...] = jnp.concatenate([
        s.reshape(_NROW, 1, 128),
        x1.reshape(_NROW, 1, 128), y1.reshape(_NROW, 1, 128),
        x2.reshape(_NROW, 1, 128), y2.reshape(_NROW, 1, 128),
        posm.reshape(_NROW, 1, 128), zrow, zrow], axis=1)

    tio = _f32(_iota2((1, _KP), 1))
    tio_c = _f32(_iota2((_KP, 1), 0))
    rowmask = _f32(_iota2((8, 1), 0) < 5)

    def row_body(r, acc):
        blk = val_ref[pl.ds(r, 1)].reshape(8, 128)
        pos_row = blk[5:6, :]
        oh = _f32(pos_row == tio_c)
        v8 = blk * rowmask
        return acc + lax.dot_general(v8, oh, (((1,), (1,)), ((), ())),
                                     preferred_element_type=f32, precision=lax.Precision.HIGHEST)

    c8 = lax.fori_loop(0, _NROW, row_body, jnp.zeros((8, _KP), f32))

    score_r = c8[0:1, :]
    score_c = _row2col(score_r)
    slot_r = _iota2((1, _KP), 1)
    slot_c = _iota2((_KP, 1), 0)
    valid_slot_c = slot_c < _K_PRE
    valid_slot_r = slot_r < _K_PRE
    beats = valid_slot_r & ((score_r > score_c) |
                            ((score_r == score_c) & (slot_r < slot_c)))
    rank_c = jnp.sum(jnp.where(beats, 1.0, 0.0).astype(f32),
                     axis=1, keepdims=True)
    rank_m = jnp.where(valid_slot_c, rank_c, 2.0e6)
    ohb = _f32(rank_m == tio)
    s8 = lax.dot_general(c8, ohb, (((1,), (0,)), ((), ())),
                         preferred_element_type=f32, precision=lax.Precision.HIGHEST)

    wimg = bounds_ref[0]
    himg = bounds_ref[1]
    sc_r = s8[0:1, :]
    x1r = jnp.clip(s8[1:2, :], 0.0, wimg)
    y1r = jnp.clip(s8[2:3, :], 0.0, himg)
    x2r = jnp.clip(s8[3:4, :], 0.0, wimg)
    y2r = jnp.clip(s8[4:5, :], 0.0, himg)
    validb_r = ((x2r - x1r >= _MIN_SIZE) & (y2r - y1r >= _MIN_SIZE) &
                (slot_r < _K_PRE))
    valid_r = jnp.where(validb_r, 1.0, 0.0).astype(f32)

    cols = _tr(jnp.concatenate([x1r, y1r, x2r, y2r, valid_r], axis=0))
    x1c = cols[:, 0:1]
    y1c = cols[:, 1:2]
    x2c = cols[:, 2:3]
    y2c = cols[:, 3:4]
    valid_c = cols[:, 4:5]

    areas_r = (x2r - x1r) * (y2r - y1r)
    areas_c = (x2c - x1c) * (y2c - y1c)

    xx1 = jnp.maximum(x1c, x1r)
    yy1 = jnp.maximum(y1c, y1r)
    xx2 = jnp.minimum(x2c, x2r)
    yy2 = jnp.minimum(y2c, y2r)
    ww = jnp.maximum(0.0, xx2 - xx1)
    hh = jnp.maximum(0.0, yy2 - yy1)
    inter = ww * hh
    iou = inter / (areas_r + areas_c - inter + 1e-12)
    mt = jnp.where((iou > _NMS_THR) & (slot_r < slot_c), 1.0, 0.0).astype(f32)
    m = _tr(mt)

    def nms_cond(carry):
        _, changed, it = carry
        return changed & (it < 600)

    def nms_body(carry):
        keep_r, _, it = carry
        s_col = jnp.sum(mt * keep_r, axis=1, keepdims=True)
        k_col = valid_c * jnp.where(s_col == 0.0, 1.0, 0.0).astype(f32)
        s_row = jnp.sum(m * k_col, axis=0, keepdims=True)
        k_row = valid_r * jnp.where(s_row == 0.0, 1.0, 0.0).astype(f32)
        changed = jnp.sum(jnp.abs(k_row - keep_r)) > 0.0
        return (k_row, changed, it + 1)

    keep_r, _, _ = lax.while_loop(
        nms_cond, nms_body, (valid_r, jnp.bool_(True), jnp.int32(0)))

    rank_f_c = jnp.sum(keep_r * _f32(slot_r <= slot_c),
                       axis=1, keepdims=True) - 1.0
    keep_c = _row2col(keep_r)
    rank_fm = jnp.where(keep_c > 0.0, rank_f_c, 2.0e6)
    oio = _f32(_iota2((1, _OUTP), 1))
    ohf = _f32(rank_fm == oio)
    dets8 = jnp.concatenate([
        x1r, y1r, x2r, y2r, sc_r, jnp.zeros((3, _KP), f32)], axis=0)
    out_ref[...] = lax.dot_general(ohf, dets8, (((0,), (1,)), ((), ())),
                                   preferred_element_type=f32, precision=lax.Precision.HIGHEST)


def _run(s2, dx2, dy2, dw2, dh2, aw2, ah2, ax2, ay2, bounds):
    return pl.pallas_call(
        _body,
        out_shape=jax.ShapeDtypeStruct((_OUTP, 8), jnp.float32),
        in_specs=[pl.BlockSpec(memory_space=pltpu.VMEM)] * 9 +
                 [pl.BlockSpec(memory_space=pltpu.SMEM)],
        out_specs=pl.BlockSpec(memory_space=pltpu.VMEM),
        scratch_shapes=[
            pltpu.VMEM((_NROW, 8, 128), jnp.float32),
        ],
    )(s2, dx2, dy2, dw2, dh2, aw2, ah2, ax2, ay2, bounds)


def _anchor_params(fmap_h, fmap_w, img_h, img_w, side):
    stride_y = img_h / fmap_h
    stride_x = img_w / fmap_w
    ws, hs = [], []
    for sscale in _SCALES:
        for r in _RATIOS:
            ws.append(sscale * math.sqrt(r))
            hs.append(sscale / math.sqrt(r))
    ws = jnp.asarray(ws, jnp.float32)
    hs = jnp.asarray(hs, jnp.float32)
    base = jnp.stack([-ws / 2, -hs / 2, ws / 2, hs / 2], axis=1)
    cy = (jnp.arange(side, dtype=jnp.float32) + 0.5) * stride_y
    cx = (jnp.arange(side, dtype=jnp.float32) + 0.5) * stride_x
    cxg, cyg = jnp.meshgrid(cx, cy)
    shifts = jnp.stack([cxg, cyg, cxg, cyg], axis=-1).reshape(-1, 1, 4)
    anchors = (shifts + base.reshape(1, -1, 4)).reshape(-1, 4)
    aw = anchors[:, 2] - anchors[:, 0]
    ah = anchors[:, 3] - anchors[:, 1]
    acx = anchors[:, 0] + 0.5 * aw
    acy = anchors[:, 1] + 0.5 * ah
    return aw, ah, acx, acy


def _pad2(v, fill):
    return jnp.concatenate(
        [v, jnp.full((_NPAD - _N,), fill, jnp.float32)]).reshape(_NROW, 128)


def kernel(cls_logits, reg_deltas, fmap_h, fmap_w, img_h, img_w):
    num_anchors = len(_SCALES) * len(_RATIOS)
    cells = cls_logits.shape[1] // num_anchors
    side = math.isqrt(cells)
    scores = jax.nn.sigmoid(cls_logits).reshape(-1)
    offs = reg_deltas.reshape(-1, 4)
    aw, ah, acx, acy = _anchor_params(fmap_h, fmap_w, img_h, img_w, side)
    s2 = _pad2(scores, -1.0)
    dx2 = _pad2(offs[:, 0], 0.0)
    dy2 = _pad2(offs[:, 1], 0.0)
    dw2 = _pad2(offs[:, 2], 0.0)
    dh2 = _pad2(offs[:, 3], 0.0)
    aw2 = _pad2(aw, 1.0)
    ah2 = _pad2(ah, 1.0)
    ax2 = _pad2(acx, 0.0)
    ay2 = _pad2(acy, 0.0)
    bounds = jnp.stack([jnp.asarray(img_w).astype(jnp.float32),
                        jnp.asarray(img_h).astype(jnp.float32)])
    res = _run(s2, dx2, dy2, dw2, dh2, aw2, ah2, ax2, ay2, bounds)
    return res[:_K_POST, :5]
